# Initial kernel scaffold; baseline (speedup 1.0000x reference)
import jax
import jax.numpy as jnp
from jax.experimental import pallas as pl


def kernel(x, distances, indices, eps, k_param):
    raise NotImplementedError("write your pallas kernel here")


# speedup vs baseline: 118.6617x; 118.6617x over previous
import jax
import jax.numpy as jnp
from jax import lax
from jax.experimental import pallas as pl
from jax.experimental.pallas import tpu as pltpu
from jax.experimental.pallas import tpu_sc as plsc

_L = 16
_C = 112


def _build(n, kk):
    info = plsc.get_sparse_core_info()
    nc, ns = info.num_cores, info.num_subcores
    nw = nc * ns
    cpw = -(-n // (nw * _C))
    np_ = nw * _C * cpw
    rows_w = _C * cpw
    blk = _C * kk
    mesh = plsc.VectorSubcoreMesh(core_axis_name="c", subcore_axis_name="s")

    def pass1_body(dist_hbm, consts_hbm, d_hbm, dist_v, dout_v, cvec_v):
        wid = lax.axis_index("c") * ns + lax.axis_index("s")
        chunk0 = wid * cpw
        pltpu.sync_copy(consts_hbm, cvec_v)
        cneg = cvec_v[...]

        def chunk(ci, carry):
            cg = chunk0 + ci
            pltpu.sync_copy(dist_hbm.at[pl.ds(cg * blk, blk)], dist_v)

            def group(g, c2):
                den = jnp.zeros((_L,), jnp.float32)
                for k in range(kk):
                    d16 = dist_v[pl.ds(k * _C + g * _L, _L)]
                    den = den + jnp.exp(d16 * cneg)
                dout_v[pl.ds(g * _L, _L)] = den
                return c2

            lax.fori_loop(0, _C // _L, group, 0)
            pltpu.sync_copy(dout_v, d_hbm.at[pl.ds(cg * _C, _C)])
            return carry

        lax.fori_loop(0, cpw, chunk, 0)

    pass1 = pl.kernel(
        pass1_body,
        out_type=jax.ShapeDtypeStruct((np_,), jnp.float32),
        mesh=mesh,
        scratch_types=[
            pltpu.VMEM((blk,), jnp.float32),
            pltpu.VMEM((_C,), jnp.float32),
            pltpu.VMEM((_L,), jnp.float32),
        ],
    )

    def pass2_body(dist_hbm, idx_hbm, x_hbm, d_hbm, consts_hbm, out_hbm,
                   dist_v, idx_v, dg_v, xg_v, xrow_v, cvec_v, outv_v, sem):
        wid = lax.axis_index("c") * ns + lax.axis_index("s")
        chunk0 = wid * cpw
        pltpu.sync_copy(consts_hbm, cvec_v)
        cneg = cvec_v[...]
        zero = jnp.zeros((_L,), jnp.float32)

        def chunk(ci, carry):
            cg = chunk0 + ci
            pltpu.sync_copy(dist_hbm.at[pl.ds(cg * blk, blk)], dist_v)
            pltpu.sync_copy(idx_hbm.at[pl.ds(cg * blk, blk)], idx_v)
            pltpu.sync_copy(x_hbm.at[pl.ds(cg * _C, _C)], xrow_v)
            cp1 = pltpu.async_copy(d_hbm.at[idx_v], dg_v, sem)
            cp2 = pltpu.async_copy(x_hbm.at[idx_v], xg_v, sem)
            cp1.wait()
            cp2.wait()

            def group(g, c2):
                al, ax = c2
                den = zero
                num = zero
                for k in range(kk):
                    off = k * _C + g * _L
                    u = jnp.exp(dist_v[pl.ds(off, _L)] * cneg) / dg_v[pl.ds(off, _L)]
                    den = den + u
                    num = num + u * xg_v[pl.ds(off, _L)]
                x16 = xrow_v[pl.ds(g * _L, _L)]
                return (al + x16 * (num / den), ax + x16 * x16)

            return lax.fori_loop(0, _C // _L, group, carry)

        al, ax = lax.fori_loop(0, cpw, chunk, (zero, zero))
        outv_v[0, :] = al
        outv_v[1, :] = ax
        pltpu.sync_copy(outv_v, out_hbm.at[wid])

    pass2 = pl.kernel(
        pass2_body,
        out_type=jax.ShapeDtypeStruct((nw, 2, _L), jnp.float32),
        mesh=mesh,
        scratch_types=[
            pltpu.VMEM((blk,), jnp.float32),
            pltpu.VMEM((blk,), jnp.int32),
            pltpu.VMEM((blk,), jnp.float32),
            pltpu.VMEM((blk,), jnp.float32),
            pltpu.VMEM((_C,), jnp.float32),
            pltpu.VMEM((_L,), jnp.float32),
            pltpu.VMEM((2, _L), jnp.float32),
            pltpu.SemaphoreType.DMA,
        ],
    )

    return pass1, pass2, np_


def kernel(x, distances, indices, eps, k_param):
    n, kk = distances.shape
    pass1, pass2, np_ = _build(n, kk)
    pad = np_ - n
    nchunks = np_ // _C

    def to_kmajor(a2d):
        return a2d.reshape(nchunks, _C, kk).transpose(0, 2, 1).reshape(-1)

    idx32 = indices.astype(jnp.int32)
    dp = to_kmajor(jnp.pad(distances, ((0, pad), (0, 0))))
    pad_idx = (jnp.arange(pad * kk, dtype=jnp.int32) % n).reshape(pad, kk)
    ip = to_kmajor(jnp.concatenate([idx32, pad_idx], axis=0))
    xp = jnp.pad(x, (0, pad))
    eps32 = eps.astype(jnp.float32)
    consts = jnp.full((_L,), -1.0, jnp.float32) / eps32

    d_table = pass1(dp, consts)
    parts = pass2(dp, ip, xp, d_table, consts)

    a = jnp.sum(parts[:, 0, :])
    b = jnp.sum(parts[:, 1, :])
    four_eps = 4.0 / eps32
    col0 = four_eps + 2.0 / (k_param.astype(jnp.float32) ** 2)
    return col0 * b - four_eps * a


# speedup vs baseline: 205.0007x; 1.7276x over previous
import jax
import jax.numpy as jnp
from jax import lax
from jax.experimental import pallas as pl
from jax.experimental.pallas import tpu as pltpu
from jax.experimental.pallas import tpu_sc as plsc

_L = 16
_C = 112


def _build(n, kk):
    info = plsc.get_sparse_core_info()
    nc, ns = info.num_cores, info.num_subcores
    nw = nc * ns
    cpw = -(-n // (nw * _C))
    np_ = nw * _C * cpw
    rows_w = _C * cpw
    blk = _C * kk
    mesh = plsc.VectorSubcoreMesh(core_axis_name="c", subcore_axis_name="s")

    def pass1_body(dist_hbm, consts_hbm, d_hbm, dist_v, dout_v, cvec_v):
        wid = lax.axis_index("c") * ns + lax.axis_index("s")
        chunk0 = wid * cpw
        pltpu.sync_copy(consts_hbm, cvec_v)
        cneg = cvec_v[...]

        def chunk(ci, carry):
            cg = chunk0 + ci
            pltpu.sync_copy(dist_hbm.at[pl.ds(cg * blk, blk)], dist_v)

            def group(g, c2):
                den = jnp.zeros((_L,), jnp.float32)
                for k in range(kk):
                    d16 = dist_v[pl.ds(k * _C + g * _L, _L)]
                    den = den + jnp.exp(d16 * cneg)
                dout_v[pl.ds(g * _L, _L)] = den
                return c2

            lax.fori_loop(0, _C // _L, group, 0)
            pltpu.sync_copy(dout_v, d_hbm.at[pl.ds(cg * _C, _C)])
            return carry

        lax.fori_loop(0, cpw, chunk, 0)

    pass1 = pl.kernel(
        pass1_body,
        out_type=jax.ShapeDtypeStruct((np_,), jnp.float32),
        mesh=mesh,
        scratch_types=[
            pltpu.VMEM((blk,), jnp.float32),
            pltpu.VMEM((_C,), jnp.float32),
            pltpu.VMEM((_L,), jnp.float32),
        ],
    )

    def pass2_body(dist_hbm, idx_hbm, x_hbm, d_hbm, consts_hbm, out_hbm,
                   dist_v, idx_v, dg_v, xg_v, xrow_v, cvec_v, outv_v,
                   d_sh, x_sh, sem):
        sid = lax.axis_index("s")
        wid = lax.axis_index("c") * ns + sid
        chunk0 = wid * cpw
        @pl.when(sid == 0)
        def _stage():
            pltpu.sync_copy(d_hbm, d_sh)
            pltpu.sync_copy(x_hbm, x_sh)
        plsc.subcore_barrier()
        pltpu.sync_copy(consts_hbm, cvec_v)
        cneg = cvec_v[...]
        zero = jnp.zeros((_L,), jnp.float32)

        def chunk(ci, carry):
            cg = chunk0 + ci
            pltpu.sync_copy(dist_hbm.at[pl.ds(cg * blk, blk)], dist_v)
            pltpu.sync_copy(idx_hbm.at[pl.ds(cg * blk, blk)], idx_v)
            pltpu.sync_copy(x_hbm.at[pl.ds(cg * _C, _C)], xrow_v)
            cp1 = pltpu.async_copy(d_sh.at[idx_v], dg_v, sem)
            cp2 = pltpu.async_copy(x_sh.at[idx_v], xg_v, sem)
            cp1.wait()
            cp2.wait()

            def group(g, c2):
                al, ax = c2
                den = zero
                num = zero
                for k in range(kk):
                    off = k * _C + g * _L
                    u = jnp.exp(dist_v[pl.ds(off, _L)] * cneg) / dg_v[pl.ds(off, _L)]
                    den = den + u
                    num = num + u * xg_v[pl.ds(off, _L)]
                x16 = xrow_v[pl.ds(g * _L, _L)]
                return (al + x16 * (num / den), ax + x16 * x16)

            return lax.fori_loop(0, _C // _L, group, carry)

        al, ax = lax.fori_loop(0, cpw, chunk, (zero, zero))
        outv_v[0, :] = al
        outv_v[1, :] = ax
        pltpu.sync_copy(outv_v, out_hbm.at[wid])

    pass2 = pl.kernel(
        pass2_body,
        out_type=jax.ShapeDtypeStruct((nw, 2, _L), jnp.float32),
        mesh=mesh,
        scratch_types=[
            pltpu.VMEM((blk,), jnp.float32),
            pltpu.VMEM((blk,), jnp.int32),
            pltpu.VMEM((blk,), jnp.float32),
            pltpu.VMEM((blk,), jnp.float32),
            pltpu.VMEM((_C,), jnp.float32),
            pltpu.VMEM((_L,), jnp.float32),
            pltpu.VMEM((2, _L), jnp.float32),
            pltpu.VMEM_SHARED((np_,), jnp.float32),
            pltpu.VMEM_SHARED((np_,), jnp.float32),
            pltpu.SemaphoreType.DMA,
        ],
    )

    return pass1, pass2, np_


def kernel(x, distances, indices, eps, k_param):
    n, kk = distances.shape
    pass1, pass2, np_ = _build(n, kk)
    pad = np_ - n
    nchunks = np_ // _C

    def to_kmajor(a2d):
        return a2d.reshape(nchunks, _C, kk).transpose(0, 2, 1).reshape(-1)

    idx32 = indices.astype(jnp.int32)
    dp = to_kmajor(jnp.pad(distances, ((0, pad), (0, 0))))
    pad_idx = (jnp.arange(pad * kk, dtype=jnp.int32) % n).reshape(pad, kk)
    ip = to_kmajor(jnp.concatenate([idx32, pad_idx], axis=0))
    xp = jnp.pad(x, (0, pad))
    eps32 = eps.astype(jnp.float32)
    consts = jnp.full((_L,), -1.0, jnp.float32) / eps32

    d_table = pass1(dp, consts)
    parts = pass2(dp, ip, xp, d_table, consts)

    a = jnp.sum(parts[:, 0, :])
    b = jnp.sum(parts[:, 1, :])
    four_eps = 4.0 / eps32
    col0 = four_eps + 2.0 / (k_param.astype(jnp.float32) ** 2)
    return col0 * b - four_eps * a


# speedup vs baseline: 246.9960x; 1.2049x over previous
import jax
import jax.numpy as jnp
from jax import lax
from jax.experimental import pallas as pl
from jax.experimental.pallas import tpu as pltpu
from jax.experimental.pallas import tpu_sc as plsc

_L = 16
_C = 112


def _build(n, kk):
    info = plsc.get_sparse_core_info()
    nc, ns = info.num_cores, info.num_subcores
    nw = nc * ns
    cpw = -(-n // (nw * _C))
    np_ = nw * _C * cpw
    blk = _C * kk
    mesh = plsc.VectorSubcoreMesh(core_axis_name="c", subcore_axis_name="s")
    i32 = jnp.int32

    def _cst(v):
        return jnp.full((_L,), v, i32)

    def pass1_body(dist_hbm, x_hbm, consts_hbm, wtab_hbm,
                   dist_v, xrow_v, wout_v, cvec_v):
        wid = lax.axis_index("c") * ns + lax.axis_index("s")
        chunk0 = wid * cpw
        pltpu.sync_copy(consts_hbm, cvec_v)
        cneg = cvec_v[...]

        def chunk(ci, carry):
            cg = chunk0 + ci
            pltpu.sync_copy(dist_hbm.at[pl.ds(cg * blk, blk)], dist_v)
            pltpu.sync_copy(x_hbm.at[pl.ds(cg * _C, _C)], xrow_v)

            def group(g, c2):
                den = jnp.zeros((_L,), jnp.float32)
                for k in range(kk):
                    d16 = dist_v[pl.ds(k * _C + g * _L, _L)]
                    den = den + jnp.exp(d16 * cneg)
                q = 1.0 / den
                p = xrow_v[pl.ds(g * _L, _L)] / den
                qi = lax.bitcast_convert_type(q, i32)
                pi = lax.bitcast_convert_type(p, i32)
                qtop = lax.shift_right_logical(qi + _cst(0x8000), _cst(16))
                ptop = (pi + _cst(0x8000)) & _cst(-65536)
                wout_v[pl.ds(g * _L, _L)] = lax.bitcast_convert_type(
                    ptop | qtop, jnp.float32)
                return c2

            lax.fori_loop(0, _C // _L, group, 0)
            pltpu.sync_copy(wout_v, wtab_hbm.at[pl.ds(cg * _C, _C)])
            return carry

        lax.fori_loop(0, cpw, chunk, 0)

    pass1 = pl.kernel(
        pass1_body,
        out_type=jax.ShapeDtypeStruct((np_,), jnp.float32),
        mesh=mesh,
        scratch_types=[
            pltpu.VMEM((blk,), jnp.float32),
            pltpu.VMEM((_C,), jnp.float32),
            pltpu.VMEM((_C,), jnp.float32),
            pltpu.VMEM((_L,), jnp.float32),
        ],
    )

    def pass2_body(dist_hbm, idx_hbm, x_hbm, wtab_hbm, consts_hbm, out_hbm,
                   dist_v0, dist_v1, idx_v0, idx_v1, wg_v0, wg_v1,
                   xrow_v0, xrow_v1, cvec_v, outv_v, w_sh, sem0, sem1):
        sid = lax.axis_index("s")
        wid = lax.axis_index("c") * ns + sid
        chunk0 = wid * cpw
        per_sub = np_ // ns
        pltpu.sync_copy(wtab_hbm.at[pl.ds(sid * per_sub, per_sub)],
                        w_sh.at[pl.ds(sid * per_sub, per_sub)])
        plsc.subcore_barrier()
        pltpu.sync_copy(consts_hbm, cvec_v)
        cneg = cvec_v[...]
        zero = jnp.zeros((_L,), jnp.float32)
        dist_b = (dist_v0, dist_v1)
        idx_b = (idx_v0, idx_v1)
        wg_b = (wg_v0, wg_v1)
        xrow_b = (xrow_v0, xrow_v1)
        sem_b = (sem0, sem1)

        def load_linear(c, b):
            cg = chunk0 + c
            pltpu.sync_copy(dist_hbm.at[pl.ds(cg * blk, blk)], dist_b[b])
            pltpu.sync_copy(idx_hbm.at[pl.ds(cg * blk, blk)], idx_b[b])
            pltpu.sync_copy(x_hbm.at[pl.ds(cg * _C, _C)], xrow_b[b])

        load_linear(0, 0)
        pltpu.async_copy(w_sh.at[idx_v0], wg_v0, sem0)

        def slot(c, b, carry):
            @pl.when(c + 1 < cpw)
            def _prefetch():
                load_linear(c + 1, 1 - b)
                pltpu.async_copy(w_sh.at[idx_b[1 - b]], wg_b[1 - b],
                                 sem_b[1 - b])
            pltpu.make_async_copy(w_sh.at[idx_b[b]], wg_b[b], sem_b[b]).wait()
            dist_v = dist_b[b]
            wg_v = wg_b[b]
            xrow_v = xrow_b[b]

            def group(g, c2):
                al, ax = c2
                den = zero
                num = zero
                for k in range(kk):
                    off = k * _C + g * _L
                    e16 = jnp.exp(dist_v[pl.ds(off, _L)] * cneg)
                    w16 = lax.bitcast_convert_type(wg_v[pl.ds(off, _L)], jnp.int32)
                    q16 = lax.bitcast_convert_type(
                        lax.shift_left(w16, _cst(16)), jnp.float32)
                    p16 = lax.bitcast_convert_type(
                        w16 & _cst(-65536), jnp.float32)
                    den = den + e16 * q16
                    num = num + e16 * p16
                x16 = xrow_v[pl.ds(g * _L, _L)]
                return (al + x16 * (num / den), ax + x16 * x16)

            return lax.fori_loop(0, _C // _L, group, carry)

        def pair(ci2, carry):
            carry = slot(ci2 * 2, 0, carry)
            return slot(ci2 * 2 + 1, 1, carry)

        al, ax = lax.fori_loop(0, cpw // 2, pair, (zero, zero))
        outv_v[0, :] = al
        outv_v[1, :] = ax
        pltpu.sync_copy(outv_v, out_hbm.at[wid])

    pass2 = pl.kernel(
        pass2_body,
        out_type=jax.ShapeDtypeStruct((nw, 2, _L), jnp.float32),
        mesh=mesh,
        scratch_types=[
            pltpu.VMEM((blk,), jnp.float32),
            pltpu.VMEM((blk,), jnp.float32),
            pltpu.VMEM((blk,), jnp.int32),
            pltpu.VMEM((blk,), jnp.int32),
            pltpu.VMEM((blk,), jnp.float32),
            pltpu.VMEM((blk,), jnp.float32),
            pltpu.VMEM((_C,), jnp.float32),
            pltpu.VMEM((_C,), jnp.float32),
            pltpu.VMEM((_L,), jnp.float32),
            pltpu.VMEM((2, _L), jnp.float32),
            pltpu.VMEM_SHARED((np_,), jnp.float32),
            pltpu.SemaphoreType.DMA,
            pltpu.SemaphoreType.DMA,
        ],
    )

    return pass1, pass2, np_


def kernel(x, distances, indices, eps, k_param):
    n, kk = distances.shape
    pass1, pass2, np_ = _build(n, kk)
    pad = np_ - n
    nchunks = np_ // _C

    def to_kmajor(a2d):
        return a2d.reshape(nchunks, _C, kk).transpose(0, 2, 1).reshape(-1)

    idx32 = indices.astype(jnp.int32)
    dp = to_kmajor(jnp.pad(distances, ((0, pad), (0, 0))))
    pad_idx = (jnp.arange(pad * kk, dtype=jnp.int32) % n).reshape(pad, kk)
    ip = to_kmajor(jnp.concatenate([idx32, pad_idx], axis=0))
    xp = jnp.pad(x, (0, pad))
    eps32 = eps.astype(jnp.float32)
    consts = jnp.full((_L,), -1.0, jnp.float32) / eps32

    wtab = pass1(dp, xp, consts)
    parts = pass2(dp, ip, xp, wtab, consts)

    a = jnp.sum(parts[:, 0, :])
    b = jnp.sum(parts[:, 1, :])
    four_eps = 4.0 / eps32
    col0 = four_eps + 2.0 / (k_param.astype(jnp.float32) ** 2)
    return col0 * b - four_eps * a


# speedup vs baseline: 339.0935x; 1.3729x over previous
import jax
import jax.numpy as jnp
from jax import lax
from jax.experimental import pallas as pl
from jax.experimental.pallas import tpu as pltpu
from jax.experimental.pallas import tpu_sc as plsc

_L = 16
_C = 448


def _build(n, kk):
    info = plsc.get_sparse_core_info()
    nc, ns = info.num_cores, info.num_subcores
    nw = nc * ns
    cpw = -(-n // (nw * _C))
    np_ = nw * _C * cpw
    blk = _C * kk
    mesh = plsc.VectorSubcoreMesh(core_axis_name="c", subcore_axis_name="s")
    i32 = jnp.int32

    def _cst(v):
        return jnp.full((_L,), v, i32)

    def _f32(v_i32):
        return lax.bitcast_convert_type(v_i32, jnp.float32)

    def pass1_body(dist_hbm, x_hbm, consts_hbm, wtab_hbm,
                   dist_v0, dist_v1, xrow_v0, xrow_v1, wout_v, cvec_v,
                   sem0, sem1):
        wid = lax.axis_index("c") * ns + lax.axis_index("s")
        chunk0 = wid * cpw
        pltpu.sync_copy(consts_hbm, cvec_v)
        cneg = cvec_v[...]
        dist_b = (dist_v0, dist_v1)
        xrow_b = (xrow_v0, xrow_v1)
        sem_b = (sem0, sem1)

        def fire(c, b):
            cg = chunk0 + c
            pltpu.async_copy(dist_hbm.at[pl.ds(cg * blk, blk)], dist_b[b],
                             sem_b[b])
            pltpu.async_copy(x_hbm.at[pl.ds(cg * _C, _C)], xrow_b[b],
                             sem_b[b])

        def drain(c, b):
            cg = chunk0 + c
            pltpu.make_async_copy(dist_hbm.at[pl.ds(cg * blk, blk)],
                                  dist_b[b], sem_b[b]).wait()
            pltpu.make_async_copy(x_hbm.at[pl.ds(cg * _C, _C)],
                                  xrow_b[b], sem_b[b]).wait()

        fire(0, 0)

        def slot(c, b, carry):
            @pl.when(c + 1 < cpw)
            def _prefetch():
                fire(c + 1, 1 - b)
            drain(c, b)
            dist_v = dist_b[b]
            xrow_v = xrow_b[b]

            def group(g, c2):
                den = jnp.zeros((_L,), jnp.float32)
                for k in range(kk):
                    d16 = dist_v[pl.ds(k * _C + g * _L, _L)]
                    den = den + jnp.exp(d16 * cneg)
                q = 1.0 / den
                p = xrow_v[pl.ds(g * _L, _L)] / den
                qi = lax.bitcast_convert_type(q, i32)
                pi = lax.bitcast_convert_type(p, i32)
                qtop = lax.shift_right_logical(qi + _cst(0x8000), _cst(16))
                ptop = (pi + _cst(0x8000)) & _cst(-65536)
                wout_v[pl.ds(g * _L, _L)] = _f32(ptop | qtop)
                return c2

            lax.fori_loop(0, _C // _L, group, 0)
            cg = chunk0 + c
            pltpu.sync_copy(wout_v, wtab_hbm.at[pl.ds(cg * _C, _C)])
            return carry

        def pair(ci2, carry):
            slot(ci2 * 2, 0, carry)
            return slot(ci2 * 2 + 1, 1, carry)

        lax.fori_loop(0, cpw // 2, pair, 0)
        if cpw % 2:
            slot(cpw - 1, 0, 0)

    pass1 = pl.kernel(
        pass1_body,
        out_type=jax.ShapeDtypeStruct((np_,), jnp.float32),
        mesh=mesh,
        scratch_types=[
            pltpu.VMEM((blk,), jnp.float32),
            pltpu.VMEM((blk,), jnp.float32),
            pltpu.VMEM((_C,), jnp.float32),
            pltpu.VMEM((_C,), jnp.float32),
            pltpu.VMEM((_C,), jnp.float32),
            pltpu.VMEM((_L,), jnp.float32),
            pltpu.SemaphoreType.DMA,
            pltpu.SemaphoreType.DMA,
        ],
    )

    def pass2_body(dist_hbm, idx_hbm, x_hbm, wtab_hbm, consts_hbm, out_hbm,
                   dist_v0, dist_v1, idx_v0, idx_v1, wg_v0, wg_v1,
                   xrow_v0, xrow_v1, cvec_v, outv_v, w_sh, sem0, sem1):
        sid = lax.axis_index("s")
        wid = lax.axis_index("c") * ns + sid
        chunk0 = wid * cpw
        per_sub = np_ // ns
        pltpu.sync_copy(wtab_hbm.at[pl.ds(sid * per_sub, per_sub)],
                        w_sh.at[pl.ds(sid * per_sub, per_sub)])
        plsc.subcore_barrier()
        pltpu.sync_copy(consts_hbm, cvec_v)
        cneg = cvec_v[...]
        zero = jnp.zeros((_L,), jnp.float32)
        dist_b = (dist_v0, dist_v1)
        idx_b = (idx_v0, idx_v1)
        wg_b = (wg_v0, wg_v1)
        xrow_b = (xrow_v0, xrow_v1)
        sem_b = (sem0, sem1)

        def load_linear(c, b):
            cg = chunk0 + c
            pltpu.sync_copy(dist_hbm.at[pl.ds(cg * blk, blk)], dist_b[b])
            pltpu.sync_copy(idx_hbm.at[pl.ds(cg * blk, blk)], idx_b[b])
            pltpu.sync_copy(x_hbm.at[pl.ds(cg * _C, _C)], xrow_b[b])

        load_linear(0, 0)
        pltpu.async_copy(w_sh.at[idx_v0], wg_v0, sem0)

        def slot(c, b, carry):
            @pl.when(c + 1 < cpw)
            def _prefetch():
                load_linear(c + 1, 1 - b)
                pltpu.async_copy(w_sh.at[idx_b[1 - b]], wg_b[1 - b],
                                 sem_b[1 - b])
            pltpu.make_async_copy(w_sh.at[idx_b[b]], wg_b[b], sem_b[b]).wait()
            dist_v = dist_b[b]
            wg_v = wg_b[b]
            xrow_v = xrow_b[b]

            def group(g, c2):
                al, ax = c2
                den = zero
                num = zero
                for k in range(kk):
                    off = k * _C + g * _L
                    e16 = jnp.exp(dist_v[pl.ds(off, _L)] * cneg)
                    w16 = lax.bitcast_convert_type(wg_v[pl.ds(off, _L)], i32)
                    q16 = _f32(lax.shift_left(w16, _cst(16)))
                    p16 = _f32(w16 & _cst(-65536))
                    den = den + e16 * q16
                    num = num + e16 * p16
                x16 = xrow_v[pl.ds(g * _L, _L)]
                return (al + x16 * (num / den), ax + x16 * x16)

            return lax.fori_loop(0, _C // _L, group, carry)

        def pair(ci2, carry):
            carry = slot(ci2 * 2, 0, carry)
            return slot(ci2 * 2 + 1, 1, carry)

        al, ax = lax.fori_loop(0, cpw // 2, pair, (zero, zero))
        if cpw % 2:
            al, ax = slot(cpw - 1, 0, (al, ax))
        outv_v[0, :] = al
        outv_v[1, :] = ax
        pltpu.sync_copy(outv_v, out_hbm.at[wid])

    pass2 = pl.kernel(
        pass2_body,
        out_type=jax.ShapeDtypeStruct((nw, 2, _L), jnp.float32),
        mesh=mesh,
        scratch_types=[
            pltpu.VMEM((blk,), jnp.float32),
            pltpu.VMEM((blk,), jnp.float32),
            pltpu.VMEM((blk,), i32),
            pltpu.VMEM((blk,), i32),
            pltpu.VMEM((blk,), jnp.float32),
            pltpu.VMEM((blk,), jnp.float32),
            pltpu.VMEM((_C,), jnp.float32),
            pltpu.VMEM((_C,), jnp.float32),
            pltpu.VMEM((_L,), jnp.float32),
            pltpu.VMEM((2, _L), jnp.float32),
            pltpu.VMEM_SHARED((np_,), jnp.float32),
            pltpu.SemaphoreType.DMA,
            pltpu.SemaphoreType.DMA,
        ],
    )

    return pass1, pass2, np_


def kernel(x, distances, indices, eps, k_param):
    n, kk = distances.shape
    pass1, pass2, np_ = _build(n, kk)
    pad = np_ - n
    nchunks = np_ // _C

    def to_kmajor(a2d):
        return (a2d.reshape(nchunks, _C, kk).transpose(0, 2, 1)
                .reshape(nchunks * _C * kk))

    idx32 = indices.astype(jnp.int32)
    dp = to_kmajor(jnp.pad(distances, ((0, pad), (0, 0))))
    pad_idx = (jnp.arange(pad * kk, dtype=jnp.int32) % n).reshape(pad, kk)
    ip = to_kmajor(jnp.concatenate([idx32, pad_idx], axis=0))
    xp = jnp.pad(x, (0, pad))
    eps32 = eps.astype(jnp.float32)
    consts = jnp.full((_L,), -1.0, jnp.float32) / eps32

    wtab = pass1(dp, xp, consts)
    parts = pass2(dp, ip, xp, wtab, consts)

    a = jnp.sum(parts[:, 0, :])
    b = jnp.sum(parts[:, 1, :])
    four_eps = 4.0 / eps32
    col0 = four_eps + 2.0 / (k_param.astype(jnp.float32) ** 2)
    return col0 * b - four_eps * a
